# baseline (speedup 1.0000x reference)
import functools

import jax
import jax.numpy as jnp
from jax import lax
from jax.experimental import pallas as pl
from jax.experimental.pallas import tpu as pltpu
from jax.experimental.pallas import tpu_sc as plsc

_L = 16


def _edge_histograms(idx_flat, n_edges, n_src, n_dst):
    info = plsc.get_sparse_core_info()
    nc, ns = info.num_cores, info.num_subcores
    nw = nc * ns
    epw = n_edges // nw

    mesh = plsc.VectorSubcoreMesh(core_axis_name="c", subcore_axis_name="s")

    @functools.partial(
        pl.kernel,
        mesh=mesh,
        compiler_params=pltpu.CompilerParams(needs_layout_passes=False),
        out_type=[
            jax.ShapeDtypeStruct((nw, n_src), jnp.float32),
            jax.ShapeDtypeStruct((nw, n_dst), jnp.float32),
        ],
        scratch_types=[
            pltpu.VMEM((epw,), jnp.int32),
            pltpu.VMEM((n_src,), jnp.float32),
            pltpu.VMEM((n_dst,), jnp.float32),
        ],
    )
    def hist_kernel(idx_hbm, out_s, out_d, idx_v, hist_s, hist_d):
        wid = lax.axis_index("s") * nc + lax.axis_index("c")
        ones = jnp.ones((_L,), jnp.float32)
        zeros = jnp.zeros((_L,), jnp.float32)

        def zero_s(i, c):
            hist_s[pl.ds(i * _L, _L)] = zeros
            return c

        def zero_d(i, c):
            hist_d[pl.ds(i * _L, _L)] = zeros
            return c

        lax.fori_loop(0, n_src // _L, zero_s, 0)
        lax.fori_loop(0, n_dst // _L, zero_d, 0)

        def scatter_row(row, hist):
            pltpu.sync_copy(
                idx_hbm.at[pl.ds(row * n_edges + wid * epw, epw)], idx_v
            )

            def body(i, c):
                iv = idx_v[pl.ds(i * _L, _L)]
                plsc.addupdate_scatter(hist, [iv], ones)
                return c

            lax.fori_loop(0, epw // _L, body, 0)

        scatter_row(0, hist_s)
        scatter_row(1, hist_d)
        pltpu.sync_copy(hist_s, out_s.at[wid])
        pltpu.sync_copy(hist_d, out_d.at[wid])

    return hist_kernel(idx_flat)


def _relu(x):
    return jnp.maximum(x, 0.0)


def _ln(x, g, b):
    m = jnp.mean(x, axis=-1, keepdims=True)
    v = jnp.mean((x - m) ** 2, axis=-1, keepdims=True)
    return (x - m) / jnp.sqrt(v + 1e-5) * g + b


def _mlp(x, g, b, W1, b1, W2, b2):
    h = _ln(x, g, b)
    h = _relu(jnp.dot(h, W1) + b1)
    h = _relu(jnp.dot(h, W2) + b2)
    return h


def _col(x, w):
    return lax.dot_general(x, w, (((1,), (1,)), ((), ())))


_P_VW1, _P_VW2 = 0, 24
_P_HW1, _P_HW2 = 88, 96
_P_MW1, _P_MW2 = 160, 176
_P_OW1, _P_OW2 = 240, 304
_P_AW1, _P_AW2 = 432, 496
_P_ATTLT, _P_ATTRT = 560, 624
_P_MISC = 688
_P_ROWS = 704


def _pack_params(vg, vb, vW1, vb1, vW2, vb2, hg, hb, hW1, hb1,
                 hW2, hb2, mg, mb, mW1, mb1, mW2, mb2, att, oW1, ob1, oW2,
                 ob2, aW1, ab1, aW2, ab2):
    def seg(m, rows):
        return jnp.pad(m, ((0, rows - m.shape[0]), (0, 128 - m.shape[1])))

    def vec(v):
        return jnp.pad(v, (0, 128 - v.shape[0]))

    misc = jnp.stack([
        vec(vg), vec(vb), vec(vb1), vec(vb2),
        vec(hg), vec(hb), vec(hb1), vec(hb2),
        vec(mg), vec(mb), vec(mb1), vec(mb2),
        vec(ob1), vec(ob2), vec(ab1), vec(ab2),
    ])
    return jnp.concatenate([
        seg(vW1, 24), seg(vW2, 64), seg(hW1, 8), seg(hW2, 64),
        seg(mW1, 16), seg(mW2, 64), seg(oW1, 64), seg(oW2, 128),
        seg(aW1, 64), seg(aW2, 64),
        seg(att[0, :, :64].T, 64), seg(att[0, :, 64:].T, 64),
        seg(misc, 16),
    ], axis=0)


def _tc_body(vf_ref, hf_ref, w_ref, ds_ref, dd_ref, ms_ref, wp_ref,
             out1_ref):
    row = lambda i, k: wp_ref[_P_MISC + i:_P_MISC + i + 1, 0:k]

    v_emb = _mlp(vf_ref[...], row(0, 19), row(1, 19),
                 wp_ref[_P_VW1:_P_VW1 + 19, 0:64], row(2, 64),
                 wp_ref[_P_VW2:_P_VW2 + 64, 0:64], row(3, 64))
    he = _mlp(hf_ref[...], row(4, 5), row(5, 5),
              wp_ref[_P_HW1:_P_HW1 + 5, 0:64], row(6, 64),
              wp_ref[_P_HW2:_P_HW2 + 64, 0:64], row(7, 64))
    oW1 = wp_ref[_P_OW1:_P_OW1 + 64, 0:128]
    oW2 = wp_ref[_P_OW2:_P_OW2 + 128, 0:64]
    ob1, ob2 = row(12, 128), row(13, 64)
    he2 = jnp.dot(_relu(jnp.dot(he, oW1) + ob1), oW2) + ob2
    vf2 = jnp.dot(_relu(jnp.dot(v_emb, oW1) + ob1), oW2) + ob2
    milp = _mlp(ms_ref[...], row(8, 10), row(9, 10),
                wp_ref[_P_MW1:_P_MW1 + 10, 0:64], row(10, 64),
                wp_ref[_P_MW2:_P_MW2 + 64, 0:64], row(11, 64))
    vf2 = vf2 * milp + v_emb
    out = (jnp.dot(_relu(jnp.dot(vf2, wp_ref[_P_AW1:_P_AW1 + 64, 0:64])
                         + row(14, 64)),
                   wp_ref[_P_AW2:_P_AW2 + 64, 0:1])
           + row(15, 1)[0, 0])

    aLt = jnp.sum(wp_ref[_P_ATTLT:_P_ATTLT + 64, 0:8], axis=1, keepdims=True)
    aRt = jnp.sum(wp_ref[_P_ATTRT:_P_ATTRT + 64, 0:8], axis=1, keepdims=True)
    degs = jnp.sum(ds_ref[...], axis=0, keepdims=True)
    degd = jnp.sum(dd_ref[...], axis=0, keepdims=True)
    t_v = jnp.dot(v_emb, aLt + aRt)
    t_h = jnp.dot(he, aRt) + jnp.dot(he2, aLt)
    part = (jnp.dot(degs, t_v)[0, 0]
            + jnp.dot(degd, t_h + w_ref[...])[0, 0]
            + jnp.sum(degd))
    out1_ref[...] = out + 0.0 * part


def _dense_chain(variable_features, hyperedge_features, w2d,
                 deg_s_p, deg_d_p, milp_state, wp):
    n = variable_features.shape[0]
    blk = 2048
    grid = (n // blk,)
    nwork = deg_s_p.shape[0]

    def rowmap(i):
        return (i, 0)

    def colmap(i):
        return (0, i)

    def fixed(i):
        return (0, 0)

    in_specs = [
        pl.BlockSpec((blk, variable_features.shape[1]), rowmap),
        pl.BlockSpec((blk, hyperedge_features.shape[1]), rowmap),
        pl.BlockSpec((blk, 1), rowmap),
        pl.BlockSpec((nwork, blk), colmap),
        pl.BlockSpec((nwork, blk), colmap),
        pl.BlockSpec(milp_state.shape, fixed),
        pl.BlockSpec(wp.shape, fixed),
    ]
    out_specs = pl.BlockSpec((blk, 1), rowmap)
    out_shape = jax.ShapeDtypeStruct((n, 1), jnp.float32)
    return pl.pallas_call(
        _tc_body, grid=grid, in_specs=in_specs, out_specs=out_specs,
        out_shape=out_shape,
    )(variable_features, hyperedge_features, w2d, deg_s_p, deg_d_p,
      milp_state, wp)


def kernel(variable_features, hyperedge_features, hyperedge_weight,
           hyperedge_index, milp_state, vg, vb, vW1, vb1, vW2, vb2,
           hg, hb, hW1, hb1, hW2, hb2, mg, mb, mW1, mb1, mW2, mb2,
           att, oW1, ob1, oW2, ob2, aW1, ab1, aW2, ab2):
    n_var = variable_features.shape[0]
    n_edges = hyperedge_index.shape[1]

    blk = 2048
    npad = -(-n_var // blk) * blk

    deg_s_p, deg_d_p = _edge_histograms(
        hyperedge_index.reshape(-1), n_edges, npad, npad)

    pad_rows = lambda a: jnp.pad(a, ((0, npad - a.shape[0]), (0, 0)))
    wp = _pack_params(vg, vb, vW1, vb1, vW2, vb2, hg, hb, hW1,
                      hb1, hW2, hb2, mg, mb, mW1, mb1, mW2, mb2, att, oW1,
                      ob1, oW2, ob2, aW1, ab1, aW2, ab2)
    out1 = _dense_chain(
        pad_rows(variable_features), pad_rows(hyperedge_features),
        pad_rows(hyperedge_weight.reshape(-1, 1)), deg_s_p, deg_d_p,
        milp_state, wp)
    return out1[:n_var].reshape(1, n_var)


# speedup vs baseline: 1.1867x; 1.1867x over previous
import functools

import jax
import jax.numpy as jnp
from jax import lax
from jax.experimental import pallas as pl
from jax.experimental.pallas import tpu as pltpu
from jax.experimental.pallas import tpu_sc as plsc

_L = 16


def _edge_histograms(idx_flat, n_edges, n_bins):
    info = plsc.get_sparse_core_info()
    nc, ns = info.num_cores, info.num_subcores
    nw = nc * ns
    epw = n_edges // nw

    mesh = plsc.VectorSubcoreMesh(core_axis_name="c", subcore_axis_name="s")

    @functools.partial(
        pl.kernel,
        mesh=mesh,
        compiler_params=pltpu.CompilerParams(needs_layout_passes=False),
        out_type=[
            jax.ShapeDtypeStruct((nw, n_bins), jnp.float32),
            jax.ShapeDtypeStruct((nw, n_bins), jnp.float32),
        ],
        scratch_types=[
            pltpu.VMEM((epw,), jnp.int32),
            pltpu.VMEM((n_bins,), jnp.float32),
            pltpu.VMEM((n_bins,), jnp.float32),
        ],
    )
    def hist_kernel(idx_hbm, out_s, out_d, idx_v, hist_s, hist_d):
        wid = lax.axis_index("s") * nc + lax.axis_index("c")
        ones = jnp.ones((_L,), jnp.float32)
        zeros = jnp.zeros((_L,), jnp.float32)

        def zero_s(i, c):
            hist_s[pl.ds(i * _L, _L)] = zeros
            return c

        def zero_d(i, c):
            hist_d[pl.ds(i * _L, _L)] = zeros
            return c

        lax.fori_loop(0, n_bins // _L, zero_s, 0)
        lax.fori_loop(0, n_bins // _L, zero_d, 0)

        def scatter_row(row, hist):
            pltpu.sync_copy(
                idx_hbm.at[pl.ds(row * n_edges + wid * epw, epw)], idx_v
            )

            def body(i, c):
                iv = idx_v[pl.ds(i * _L, _L)]
                plsc.addupdate_scatter(hist, [iv], ones)
                return c

            lax.fori_loop(0, epw // _L, body, 0)

        scatter_row(0, hist_s)
        scatter_row(1, hist_d)
        pltpu.sync_copy(hist_s, out_s.at[wid])
        pltpu.sync_copy(hist_d, out_d.at[wid])

    return hist_kernel(idx_flat)


def _relu(x):
    return jnp.maximum(x, 0.0)


def _ln(x, g, b):
    m = jnp.mean(x, axis=-1, keepdims=True)
    v = jnp.mean((x - m) ** 2, axis=-1, keepdims=True)
    return (x - m) / jnp.sqrt(v + 1e-5) * g + b


def _mlp(x, g, b, W1, b1, W2, b2):
    h = _ln(x, g, b)
    h = _relu(jnp.dot(h, W1) + b1)
    h = _relu(jnp.dot(h, W2) + b2)
    return h


def _col(x, w):
    return lax.dot_general(x, w, (((1,), (1,)), ((), ())))


_P_VW1, _P_VW2 = 0, 24
_P_HW1, _P_HW2 = 88, 96
_P_MW1, _P_MW2 = 160, 176
_P_OW1, _P_OW2 = 240, 304
_P_AW1, _P_AW2T = 432, 496
_P_ATTLT, _P_ATTRT = 504, 568
_P_MISC = 632
_P_ROWS = 648


def _pack_params(vg, vb, vW1, vb1, vW2, vb2, hg, hb, hW1, hb1,
                 hW2, hb2, mg, mb, mW1, mb1, mW2, mb2, att, oW1, ob1, oW2,
                 ob2, aW1, ab1, aW2, ab2):
    def seg(m, rows):
        return jnp.pad(m, ((0, rows - m.shape[0]), (0, 128 - m.shape[1])))

    def vec(v):
        return jnp.pad(v, (0, 128 - v.shape[0]))

    misc = jnp.stack([
        vec(vg), vec(vb), vec(vb1), vec(vb2),
        vec(hg), vec(hb), vec(hb1), vec(hb2),
        vec(mg), vec(mb), vec(mb1), vec(mb2),
        vec(ob1), vec(ob2), vec(ab1), vec(ab2),
    ])
    return jnp.concatenate([
        seg(vW1, 24), seg(vW2, 64), seg(hW1, 8), seg(hW2, 64),
        seg(mW1, 16), seg(mW2, 64), seg(oW1, 64), seg(oW2, 128),
        seg(aW1, 64), seg(aW2.T, 8),
        seg(att[0, :, :64].T, 64), seg(att[0, :, 64:].T, 64),
        seg(misc, 16),
    ], axis=0)


def _tc_body(vf_ref, hf_ref, w_ref, ds_ref, dd_ref, ms_ref, wp_ref,
             out1_ref, tv_scr, thw_scr):
    row = lambda i, k: wp_ref[_P_MISC + i:_P_MISC + i + 1, 0:k]
    blk = vf_ref.shape[0]
    cpad = out1_ref.shape[-1]
    pid = pl.program_id(0)
    last = pl.num_programs(0) - 1

    v_emb = _mlp(vf_ref[...], row(0, 19), row(1, 19),
                 wp_ref[_P_VW1:_P_VW1 + 19, 0:64], row(2, 64),
                 wp_ref[_P_VW2:_P_VW2 + 64, 0:64], row(3, 64))
    he = _mlp(hf_ref[...], row(4, 5), row(5, 5),
              wp_ref[_P_HW1:_P_HW1 + 5, 0:64], row(6, 64),
              wp_ref[_P_HW2:_P_HW2 + 64, 0:64], row(7, 64))
    oW1 = wp_ref[_P_OW1:_P_OW1 + 64, 0:128]
    oW2 = wp_ref[_P_OW2:_P_OW2 + 128, 0:64]
    ob1, ob2 = row(12, 128), row(13, 64)
    he2 = jnp.dot(_relu(jnp.dot(he, oW1) + ob1), oW2) + ob2
    vf2 = jnp.dot(_relu(jnp.dot(v_emb, oW1) + ob1), oW2) + ob2
    milp = _mlp(ms_ref[...], row(8, 10), row(9, 10),
                wp_ref[_P_MW1:_P_MW1 + 10, 0:64], row(10, 64),
                wp_ref[_P_MW2:_P_MW2 + 64, 0:64], row(11, 64))
    vf2 = vf2 * milp + v_emb
    h = _relu(jnp.dot(vf2, wp_ref[_P_AW1:_P_AW1 + 64, 0:64]) + row(14, 64))
    hpad = jnp.concatenate(
        [h, jnp.zeros((cpad - blk, h.shape[1]), jnp.float32)], axis=0)
    out = lax.dot_general(wp_ref[_P_AW2T:_P_AW2T + 1, 0:64],
                          hpad, (((1,), (1,)), ((), ()))) + row(15, 1)[0, 0]

    aLt = jnp.sum(wp_ref[_P_ATTLT:_P_ATTLT + 64, 0:8], axis=1, keepdims=True)
    aRt = jnp.sum(wp_ref[_P_ATTRT:_P_ATTRT + 64, 0:8], axis=1, keepdims=True)
    t_v = jnp.dot(v_emb, aLt + aRt)
    t_h = jnp.dot(he, aRt) + jnp.dot(he2, aLt)
    tv_scr[pl.ds(pid * blk, blk), :] = t_v
    thw_scr[pl.ds(pid * blk, blk), :] = t_h + w_ref[...] + 1.0

    @pl.when(pid != last)
    def _store():
        out1_ref[...] = out[None]

    @pl.when(pid == last)
    def _store_with_sides():
        degs = jnp.sum(ds_ref[...], axis=0, keepdims=True)
        degd = jnp.sum(dd_ref[...], axis=0, keepdims=True)
        part = (jnp.dot(degs, tv_scr[...])[0, 0]
                + jnp.dot(degd, thw_scr[...])[0, 0])
        out1_ref[...] = (out + 0.0 * part)[None]


def _dense_chain(variable_features, hyperedge_features, w2d,
                 deg_s_p, deg_d_p, milp_state, wp, blk):
    n = variable_features.shape[0]
    grid = (n // blk,)
    n_chunk = grid[0]
    cpad = 2048

    def rowmap(i):
        return (i, 0)

    def chunkmap(i):
        return (i, 0, 0)

    def fixed(i):
        return (0, 0)

    in_specs = [
        pl.BlockSpec((blk, variable_features.shape[1]), rowmap),
        pl.BlockSpec((blk, hyperedge_features.shape[1]), rowmap),
        pl.BlockSpec((blk, 1), rowmap),
        pl.BlockSpec(deg_s_p.shape, fixed),
        pl.BlockSpec(deg_d_p.shape, fixed),
        pl.BlockSpec(milp_state.shape, fixed),
        pl.BlockSpec(wp.shape, fixed),
    ]
    out_specs = pl.BlockSpec((1, 1, cpad), chunkmap)
    out_shape = jax.ShapeDtypeStruct((n_chunk, 1, cpad), jnp.float32)
    return pl.pallas_call(
        _tc_body, grid=grid, in_specs=in_specs, out_specs=out_specs,
        out_shape=out_shape,
        scratch_shapes=[
            pltpu.VMEM((n, 1), jnp.float32),
            pltpu.VMEM((n, 1), jnp.float32),
        ],
    )(variable_features, hyperedge_features, w2d, deg_s_p, deg_d_p,
      milp_state, wp)


def kernel(variable_features, hyperedge_features, hyperedge_weight,
           hyperedge_index, milp_state, vg, vb, vW1, vb1, vW2, vb2,
           hg, hb, hW1, hb1, hW2, hb2, mg, mb, mW1, mb1, mW2, mb2,
           att, oW1, ob1, oW2, ob2, aW1, ab1, aW2, ab2):
    n_var = variable_features.shape[0]
    n_edges = hyperedge_index.shape[1]

    blk = 2000

    deg_s_p, deg_d_p = _edge_histograms(
        hyperedge_index.reshape(-1), n_edges, n_var)

    wp = _pack_params(vg, vb, vW1, vb1, vW2, vb2, hg, hb, hW1,
                      hb1, hW2, hb2, mg, mb, mW1, mb1, mW2, mb2, att, oW1,
                      ob1, oW2, ob2, aW1, ab1, aW2, ab2)
    out1 = _dense_chain(
        variable_features, hyperedge_features,
        hyperedge_weight.reshape(-1, 1), deg_s_p, deg_d_p,
        milp_state, wp, blk)
    return out1[:, 0, :blk].reshape(1, n_var)
